# Initial kernel scaffold; baseline (speedup 1.0000x reference)
import jax
import jax.numpy as jnp
from jax.experimental import pallas as pl


def kernel(x, edge_index, batch, W1, b1, W2, b2, W3, b3, Wlin, blin):
    raise NotImplementedError("write your pallas kernel here")


# speedup vs baseline: 18.5692x; 18.5692x over previous
import functools

import jax
import jax.numpy as jnp
from jax import lax
from jax.experimental import pallas as pl
from jax.experimental.pallas import tpu as pltpu
from jax.experimental.pallas import tpu_sc as plsc

N = 10000
E = 320000
G = 64
D = 128
DOUT = 16

NC = 2
NS = 16
NW = NC * NS
EPW = E // NW
K = 125
NSTEPS = EPW // K
NP = 10240
RPT = NP // NS
ZC = 128
DEGW = 16
R = 1000
GRID = N // R

_mesh = plsc.VectorSubcoreMesh(
    core_axis_name="c", subcore_axis_name="s", num_cores=NC, num_subcores=NS)


@functools.partial(
    pl.kernel,
    out_type=jax.ShapeDtypeStruct((NC, NP, DEGW), jnp.float32),
    mesh=_mesh,
    scratch_types=[
        pltpu.VMEM((NSTEPS, K), jnp.int32),
        pltpu.VMEM((K, DEGW), jnp.float32),
        pltpu.VMEM((ZC, DEGW), jnp.float32),
        pltpu.VMEM_SHARED((NP, DEGW), jnp.float32),
    ],
)
def _deg_kernel(dst_hbm, out_hbm, dstv, rows_v, zbuf, acc):
    cid = lax.axis_index("c")
    sid = lax.axis_index("s")
    wid = cid * NS + sid

    def body0(i, _):
        zbuf[i] = jnp.zeros((DEGW,), jnp.float32)
        return 0
    lax.fori_loop(0, ZC, body0, 0)
    for r in range(RPT // ZC):
        pltpu.sync_copy(zbuf, acc.at[pl.ds(sid * RPT + r * ZC, ZC)])

    def body1(i, _):
        rows_v[i] = jnp.ones((DEGW,), jnp.float32)
        return 0
    lax.fori_loop(0, K, body1, 0)
    pltpu.sync_copy(dst_hbm.at[wid], dstv)
    plsc.subcore_barrier()

    def step(j, _):
        pltpu.sync_copy(rows_v, acc.at[dstv.at[j]], add=True)
        return 0
    lax.fori_loop(0, NSTEPS, step, 0)
    plsc.subcore_barrier()
    pltpu.sync_copy(acc.at[pl.ds(sid * RPT, RPT)],
                    out_hbm.at[cid, pl.ds(sid * RPT, RPT)])


@functools.partial(
    pl.kernel,
    out_type=jax.ShapeDtypeStruct((NC, NP, D), jnp.float32),
    mesh=_mesh,
    scratch_types=[
        pltpu.VMEM((NSTEPS, K), jnp.int32),
        pltpu.VMEM((NSTEPS, K), jnp.int32),
        pltpu.VMEM((K, D), jnp.float32),
        pltpu.VMEM_SHARED((NP, D), jnp.float32),
        pltpu.SemaphoreType.DMA,
    ],
)
def _agg_kernel(hs_hbm, src_hbm, dst_hbm, zeros_hbm, out_hbm,
                srcv, dstv, buf, acc, sem):
    cid = lax.axis_index("c")
    sid = lax.axis_index("s")
    wid = cid * NS + sid

    pltpu.sync_copy(zeros_hbm.at[pl.ds(sid * RPT, RPT)],
                    acc.at[pl.ds(sid * RPT, RPT)])
    pltpu.sync_copy(src_hbm.at[wid], srcv)
    pltpu.sync_copy(dst_hbm.at[wid], dstv)
    plsc.subcore_barrier()

    def step(j, _):
        pltpu.async_copy(hs_hbm.at[srcv.at[j]], buf, sem).wait()
        pltpu.sync_copy(buf, acc.at[dstv.at[j]], add=True)
        return 0
    lax.fori_loop(0, NSTEPS, step, 0)
    plsc.subcore_barrier()
    pltpu.sync_copy(acc.at[pl.ds(sid * RPT, RPT)],
                    out_hbm.at[cid, pl.ds(sid * RPT, RPT)])


def _dinv_of(degp_ref):
    deg = degp_ref[0, :, 0:1] + degp_ref[1, :, 0:1] + 1.0
    return lax.rsqrt(deg)


def _mm1_body(degp_ref, x_ref, w_ref, o_ref):
    o_ref[...] = _dinv_of(degp_ref) * jnp.dot(
        x_ref[...], w_ref[...], preferred_element_type=jnp.float32)


_mm1 = pl.pallas_call(
    _mm1_body,
    grid=(GRID,),
    in_specs=[
        pl.BlockSpec((NC, R, DEGW), lambda i: (0, i, 0)),
        pl.BlockSpec((R, D), lambda i: (i, 0)),
        pl.BlockSpec((D, D), lambda i: (0, 0)),
    ],
    out_specs=pl.BlockSpec((R, D), lambda i: (i, 0)),
    out_shape=jax.ShapeDtypeStruct((N, D), jnp.float32),
)


def _mix_body(degp_ref, sp_ref, hs_ref, b_ref, w_ref, o_ref):
    dinv = _dinv_of(degp_ref)
    tot = sp_ref[0] + sp_ref[1] + hs_ref[...]
    h = jnp.maximum(dinv * tot + b_ref[...], 0.0)
    o_ref[...] = dinv * jnp.dot(h, w_ref[...],
                                preferred_element_type=jnp.float32)


_mix = pl.pallas_call(
    _mix_body,
    grid=(GRID,),
    in_specs=[
        pl.BlockSpec((NC, R, DEGW), lambda i: (0, i, 0)),
        pl.BlockSpec((NC, R, D), lambda i: (0, i, 0)),
        pl.BlockSpec((R, D), lambda i: (i, 0)),
        pl.BlockSpec((1, D), lambda i: (0, 0)),
        pl.BlockSpec((D, D), lambda i: (0, 0)),
    ],
    out_specs=pl.BlockSpec((R, D), lambda i: (i, 0)),
    out_shape=jax.ShapeDtypeStruct((N, D), jnp.float32),
)


def _fin_body(degp_ref, sp_ref, hs_ref, b_ref, wl_ref, bl_ref, batch_ref,
              o_ref, acc_s, acc_c):
    i = pl.program_id(0)
    dinv = _dinv_of(degp_ref)
    tot = sp_ref[0] + sp_ref[1] + hs_ref[...]
    h = jnp.maximum(dinv * tot + b_ref[...], 0.0)
    t = jnp.tanh(jnp.dot(h, wl_ref[...], preferred_element_type=jnp.float32)
                 + bl_ref[...])
    bt = batch_ref[0, 0, :]
    oh = (lax.broadcasted_iota(jnp.int32, (G, R), 0)
          == bt[None, :]).astype(jnp.float32)
    ps = jnp.dot(oh, t, preferred_element_type=jnp.float32)
    pc = jnp.sum(oh, axis=1)[:, None]

    @pl.when(i == 0)
    def _():
        acc_s[...] = jnp.zeros_like(acc_s)
        acc_c[...] = jnp.zeros_like(acc_c)

    acc_s[...] += ps
    acc_c[...] += pc

    @pl.when(i == pl.num_programs(0) - 1)
    def _():
        o_ref[...] = acc_s[...] / jnp.maximum(acc_c[...], 1.0)


_fin = pl.pallas_call(
    _fin_body,
    grid=(GRID,),
    in_specs=[
        pl.BlockSpec((NC, R, DEGW), lambda i: (0, i, 0)),
        pl.BlockSpec((NC, R, D), lambda i: (0, i, 0)),
        pl.BlockSpec((R, D), lambda i: (i, 0)),
        pl.BlockSpec((1, D), lambda i: (0, 0)),
        pl.BlockSpec((D, DOUT), lambda i: (0, 0)),
        pl.BlockSpec((1, DOUT), lambda i: (0, 0)),
        pl.BlockSpec((1, 1, R), lambda i: (i, 0, 0)),
    ],
    out_specs=pl.BlockSpec((G, DOUT), lambda i: (0, 0)),
    out_shape=jax.ShapeDtypeStruct((G, DOUT), jnp.float32),
    scratch_shapes=[
        pltpu.VMEM((G, DOUT), jnp.float32),
        pltpu.VMEM((G, 1), jnp.float32),
    ],
)


def kernel(x, edge_index, batch, W1, b1, W2, b2, W3, b3, Wlin, blin):
    src = edge_index[0].astype(jnp.int32).reshape(NW, NSTEPS, K)
    dst = edge_index[1].astype(jnp.int32).reshape(NW, NSTEPS, K)
    zeros = jnp.zeros((NP, D), jnp.float32)
    batch3 = batch.astype(jnp.int32).reshape(GRID, 1, R)
    b1r, b2r, b3r = (b.reshape(1, D) for b in (b1, b2, b3))
    blr = blin.reshape(1, DOUT)

    degp = _deg_kernel(dst)
    hs = _mm1(degp, x, W1)
    sp = _agg_kernel(hs, src, dst, zeros)
    hs = _mix(degp, sp, hs, b1r, W2)
    sp = _agg_kernel(hs, src, dst, zeros)
    hs = _mix(degp, sp, hs, b2r, W3)
    sp = _agg_kernel(hs, src, dst, zeros)
    return _fin(degp, sp, hs, b3r, Wlin, blr, batch3)
